# baseline (speedup 1.0000x reference)
import functools

import jax
import jax.numpy as jnp
from jax import lax
from jax.experimental import pallas as pl
from jax.experimental.pallas import tpu as pltpu
from jax.experimental.pallas import tpu_sc as plsc

_NC = 2
_NS = 16
_N = 256
_D = 4 * _N
_CHUNK = 16
_NBUF = 4


def _sc_stream_kernel(x_hbm, months_hbm, ce_hbm, pe_hbm, mt_hbm, out_hbm,
                      ce_v, pe_v, midx_v, mrows_v, abuf, xbufs,
                      gsem, in_sems, out_sems):
    t = pe_v.shape[0]
    b_s = ce_v.shape[0]
    rows_total = x_hbm.shape[0]
    wid = lax.axis_index("s") * _NC + lax.axis_index("c")
    nw = _NC * _NS
    rows_per_w = rows_total // nw
    tiles_per_b = (rows_total // 2) // rows_per_w
    base = wid * rows_per_w
    b = wid // tiles_per_b

    pltpu.sync_copy(ce_hbm, ce_v)
    pltpu.sync_copy(pe_hbm, pe_v)
    pltpu.sync_copy(months_hbm.at[b], midx_v)
    pltpu.async_copy(mt_hbm.at[midx_v], mrows_v, gsem).wait()

    def build_row(r, carry):
        tt = r // b_s
        ss = r % b_s
        for v in range(_N // 16):
            sl = pl.ds(v * 16, 16)
            abuf[r, pl.ds(0 * _N + v * 16, 16)] = ce_v[ss, sl]
            abuf[r, pl.ds(1 * _N + v * 16, 16)] = pe_v[tt, sl]
            abuf[r, pl.ds(2 * _N + v * 16, 16)] = mrows_v[tt, sl]
        return carry

    lax.fori_loop(0, t * b_s, build_row, 0)

    n_chunks = rows_per_w // _CHUNK
    pattern = t * b_s

    def in_copy(c, k):
        return pltpu.make_async_copy(
            x_hbm.at[pl.ds(base + c * _CHUNK, _CHUNK)], xbufs[k], in_sems.at[k])

    def out_copy(c, k):
        return pltpu.make_async_copy(
            xbufs[k], out_hbm.at[pl.ds(base + c * _CHUNK, _CHUNK)], out_sems.at[k])

    for c in range(2):
        in_copy(c, c).start()

    def round_body(i, carry):
        c0 = i * _NBUF
        for k in range(_NBUF):
            c = c0 + k
            in_copy(c, k).wait()
            xbuf = xbufs[k]
            arow0 = (c * _CHUNK) % pattern

            def add_row(r, inner, xbuf=xbuf, arow0=arow0):
                ar = arow0 + r
                for q in range(3):
                    for v in range(_N // 16):
                        sl = pl.ds(q * _N + v * 16, 16)
                        plsc.addupdate(xbuf.at[r, sl], abuf[ar, sl])
                return inner

            lax.fori_loop(0, _CHUNK, add_row, 0)
            out_copy(c, k).start()
            kp = (k + 2) % _NBUF
            f = c + 2

            @pl.when(f < n_chunks)
            def _prefetch(f=f, kp=kp):
                @pl.when(f >= _NBUF)
                def _drain():
                    out_copy(f - _NBUF, kp).wait()

                in_copy(f, kp).start()

        return carry

    lax.fori_loop(0, n_chunks // _NBUF, round_body, 0)

    for k in range(_NBUF):
        out_copy(n_chunks - _NBUF + k, k).wait()


def kernel(sensor_tokens, timestamps, channel_embed, pos_embed, month_table):
    b, h, w, t, b_s, d = sensor_tokens.shape
    n = d // 4
    rows = b * h * w * t * b_s
    x = sensor_tokens.reshape(rows, d)
    months = timestamps[:, :, 1].astype(jnp.int32)
    pos8 = pos_embed[:t]

    runner = functools.partial(
        pl.kernel,
        out_type=jax.ShapeDtypeStruct((rows, d), jnp.float32),
        mesh=plsc.VectorSubcoreMesh(
            core_axis_name="c", subcore_axis_name="s",
            num_cores=_NC, num_subcores=_NS),
        scratch_types=[
            pltpu.VMEM((b_s, n), jnp.float32),
            pltpu.VMEM((t, n), jnp.float32),
            pltpu.VMEM((t,), jnp.int32),
            pltpu.VMEM((t, n), jnp.float32),
            pltpu.VMEM((t * b_s, 3 * n), jnp.float32),
            [pltpu.VMEM((_CHUNK, d), jnp.float32) for _ in range(_NBUF)],
            pltpu.SemaphoreType.DMA,
            pltpu.SemaphoreType.DMA((_NBUF,)),
            pltpu.SemaphoreType.DMA((_NBUF,)),
        ],
    )(_sc_stream_kernel)
    out = runner(x, months, channel_embed, pos8, month_table)
    return out.reshape(b, h, w, t, b_s, d)


# speedup vs baseline: 4.1212x; 4.1212x over previous
import functools

import jax
import jax.numpy as jnp
from jax import lax
from jax.experimental import pallas as pl
from jax.experimental.pallas import tpu as pltpu
from jax.experimental.pallas import tpu_sc as plsc

_NC = 2
_NS = 16


def _sc_month_gather(months_hbm, mt_hbm, me_hbm, midx_v, mrows_v, gsem):
    b = months_hbm.shape[0]
    wid = lax.axis_index("s") * _NC + lax.axis_index("c")

    @pl.when(wid < b)
    def _gather():
        pltpu.sync_copy(months_hbm.at[wid], midx_v)
        pltpu.async_copy(mt_hbm.at[midx_v], mrows_v, gsem).wait()
        pltpu.sync_copy(mrows_v, me_hbm.at[wid])


def _month_rows(months, month_table):
    b, t = months.shape
    n = month_table.shape[-1]
    runner = functools.partial(
        pl.kernel,
        out_type=jax.ShapeDtypeStruct((b, t, n), jnp.float32),
        mesh=plsc.VectorSubcoreMesh(
            core_axis_name="c", subcore_axis_name="s",
            num_cores=_NC, num_subcores=_NS),
        scratch_types=[
            pltpu.VMEM((t,), jnp.int32),
            pltpu.VMEM((t, n), jnp.float32),
            pltpu.SemaphoreType.DMA,
        ],
    )(_sc_month_gather)
    return runner(months, month_table)


def _tc_add_kernel(x_ref, ce_ref, pe_ref, me_ref, o_ref):
    t = pe_ref.shape[0]
    n = ce_ref.shape[-1]
    x = x_ref[...]
    ce = ce_ref[...]
    pe = pe_ref[...]
    me = me_ref[0]
    o_ref[..., 0:n] = x[..., 0:n] + ce[None, None, None, :, :]
    o_ref[..., n:2 * n] = x[..., n:2 * n] + pe[None, None, :, None, :]
    o_ref[..., 2 * n:3 * n] = x[..., 2 * n:3 * n] + me[None, None, :, None, :]
    o_ref[..., 3 * n:] = x[..., 3 * n:]


def kernel(sensor_tokens, timestamps, channel_embed, pos_embed, month_table):
    b, h, w, t, b_s, d = sensor_tokens.shape
    n = d // 4
    hw = h * w
    br = 32
    x = sensor_tokens.reshape(b, hw, t, b_s, d)
    months = timestamps[:, :, 1].astype(jnp.int32)

    me = _month_rows(months, month_table)

    out = pl.pallas_call(
        _tc_add_kernel,
        grid=(b, hw // br),
        in_specs=[
            pl.BlockSpec((1, br, t, b_s, d), lambda i, j: (i, j, 0, 0, 0)),
            pl.BlockSpec((b_s, n), lambda i, j: (0, 0)),
            pl.BlockSpec((t, n), lambda i, j: (0, 0)),
            pl.BlockSpec((1, t, n), lambda i, j: (i, 0, 0)),
        ],
        out_specs=pl.BlockSpec((1, br, t, b_s, d), lambda i, j: (i, j, 0, 0, 0)),
        out_shape=jax.ShapeDtypeStruct(x.shape, x.dtype),
        compiler_params=pltpu.CompilerParams(
            dimension_semantics=("arbitrary", "arbitrary"),
        ),
    )(x, channel_embed, pos_embed[:t], me)
    return out.reshape(b, h, w, t, b_s, d)
